# baseline (device time: 27284 ns/iter reference)
import jax
import jax.numpy as jnp
from jax import lax
from jax.experimental import pallas as pl
from jax.experimental.pallas import tpu as pltpu

N_DEV = 8
EPS = 1e-5
B = 8


def kernel(x, gamma, beta):
    m, n_local = x.shape
    n_global = n_local * N_DEV
    mb = m // B
    rb = mb // 128

    def body(x_hbm, g_ref, b_ref, out_hbm, xv, ov, comm_ref,
             in_sems, out_sems, send_sems, recv_sems):
        my_pos = lax.axis_index("i")

        in_copies = []
        for blk in range(B):
            cp = pltpu.make_async_copy(
                x_hbm.at[pl.ds(blk * mb, mb), :],
                xv.at[pl.ds(blk * mb, mb), :],
                in_sems.at[blk],
            )
            cp.start()
            in_copies.append(cp)

        barrier_sem = pltpu.get_barrier_semaphore()
        for k in range(1, N_DEV):
            peer = (my_pos + k) % N_DEV
            pl.semaphore_signal(
                barrier_sem, inc=1,
                device_id=(peer,), device_id_type=pl.DeviceIdType.MESH,
            )
        pl.semaphore_wait(barrier_sem, N_DEV - 1)

        br = lax.broadcasted_iota(jnp.int32, (mb, rb), 0)
        bi = lax.broadcasted_iota(jnp.int32, (mb, rb), 1)
        O = (br // 128 == bi).astype(jnp.float32)
        rr = lax.broadcasted_iota(jnp.int32, (mb, 128), 0)
        jj = lax.broadcasted_iota(jnp.int32, (mb, 128), 1)
        L = (rr % 128 == jj).astype(jnp.float32)

        def unpack(packed):
            q = jnp.dot(O, packed, preferred_element_type=jnp.float32)
            return jnp.sum(q * L, axis=1, keepdims=True)

        g = g_ref[...].astype(jnp.float32)
        b = b_ref[...].astype(jnp.float32)

        sends = []
        out_copies = [None, None]

        def finish(blk):
            for k in range(1, N_DEV):
                src = (my_pos + k) % N_DEV
                recv = pltpu.make_async_remote_copy(
                    src_ref=comm_ref.at[src, blk],
                    dst_ref=comm_ref.at[src, blk],
                    send_sem=send_sems.at[blk, k - 1],
                    recv_sem=recv_sems.at[src, blk],
                    device_id=(src,),
                    device_id_type=pl.DeviceIdType.MESH,
                )
                recv.wait_recv()
            tot = jnp.sum(comm_ref[:, blk], axis=0)
            mean = unpack(tot[0]) / n_global
            ex2 = unpack(tot[1]) / n_global
            inv = lax.rsqrt(ex2 - mean * mean + EPS)
            slot = blk % 2
            if out_copies[slot] is not None:
                out_copies[slot].wait()
            xb = xv[blk * mb:(blk + 1) * mb, :]
            ov[slot] = ((xb - mean) * inv * g + b).astype(ov.dtype)
            cp = pltpu.make_async_copy(
                ov.at[slot],
                out_hbm.at[pl.ds(blk * mb, mb), :],
                out_sems.at[slot],
            )
            cp.start()
            out_copies[slot] = cp

        for blk in range(B):
            in_copies[blk].wait()
            xb = xv[blk * mb:(blk + 1) * mb, :]
            s = jnp.sum(xb, axis=1)
            ss = jnp.sum(xb * xb, axis=1)
            comm_ref[my_pos, blk, 0] = s.reshape(rb, 128)
            comm_ref[my_pos, blk, 1] = ss.reshape(rb, 128)
            for k in range(1, N_DEV):
                peer = (my_pos + k) % N_DEV
                rdma = pltpu.make_async_remote_copy(
                    src_ref=comm_ref.at[my_pos, blk],
                    dst_ref=comm_ref.at[my_pos, blk],
                    send_sem=send_sems.at[blk, k - 1],
                    recv_sem=recv_sems.at[my_pos, blk],
                    device_id=(peer,),
                    device_id_type=pl.DeviceIdType.MESH,
                )
                rdma.start()
                sends.append(rdma)
            if blk >= 2:
                finish(blk - 2)
        finish(B - 2)
        finish(B - 1)

        for cp in out_copies:
            if cp is not None:
                cp.wait()
        for r in sends:
            r.wait_send()

    return pl.pallas_call(
        body,
        out_shape=jax.ShapeDtypeStruct((m, n_local), jnp.bfloat16),
        in_specs=[
            pl.BlockSpec(memory_space=pltpu.MemorySpace.HBM),
            pl.BlockSpec(memory_space=pltpu.VMEM),
            pl.BlockSpec(memory_space=pltpu.VMEM),
        ],
        out_specs=pl.BlockSpec(memory_space=pltpu.MemorySpace.HBM),
        scratch_shapes=[
            pltpu.VMEM((m, n_local), jnp.float32),
            pltpu.VMEM((2, mb, n_local), jnp.bfloat16),
            pltpu.VMEM((N_DEV, B, 2, mb // 128, 128), jnp.float32),
            pltpu.SemaphoreType.DMA((B,)),
            pltpu.SemaphoreType.DMA((2,)),
            pltpu.SemaphoreType.DMA((B, N_DEV - 1)),
            pltpu.SemaphoreType.DMA((N_DEV, B)),
        ],
        compiler_params=pltpu.CompilerParams(
            collective_id=0, vmem_limit_bytes=100 * 1024 * 1024
        ),
    )(x, gamma.reshape(1, n_local), beta.reshape(1, n_local))


# device time: 23827 ns/iter; 1.1451x vs baseline; 1.1451x over previous
import jax
import jax.numpy as jnp
from jax import lax
from jax.experimental import pallas as pl
from jax.experimental.pallas import tpu as pltpu

N_DEV = 8
EPS = 1e-5
B = 4


def kernel(x, gamma, beta):
    m, n_local = x.shape
    n_global = n_local * N_DEV
    mb = m // B
    rb = mb // 128

    def body(x_hbm, g_ref, b_ref, out_hbm, xv, ov, comm_ref,
             in_sems, out_sems, send_sems, recv_sems):
        my_pos = lax.axis_index("i")

        in_copies = []
        for blk in range(B):
            cp = pltpu.make_async_copy(
                x_hbm.at[pl.ds(blk * mb, mb), :],
                xv.at[pl.ds(blk * mb, mb), :],
                in_sems.at[blk],
            )
            cp.start()
            in_copies.append(cp)

        barrier_sem = pltpu.get_barrier_semaphore()
        for k in range(1, N_DEV):
            peer = (my_pos + k) % N_DEV
            pl.semaphore_signal(
                barrier_sem, inc=1,
                device_id=(peer,), device_id_type=pl.DeviceIdType.MESH,
            )
        pl.semaphore_wait(barrier_sem, N_DEV - 1)

        br = lax.broadcasted_iota(jnp.int32, (mb, rb), 0)
        bi = lax.broadcasted_iota(jnp.int32, (mb, rb), 1)
        O = (br // 128 == bi).astype(jnp.float32)
        rr = lax.broadcasted_iota(jnp.int32, (mb, 128), 0)
        jj = lax.broadcasted_iota(jnp.int32, (mb, 128), 1)
        L = (rr % 128 == jj).astype(jnp.float32)

        def unpack(packed):
            q = jnp.dot(O, packed, preferred_element_type=jnp.float32)
            return jnp.sum(q * L, axis=1, keepdims=True)

        g = g_ref[...].astype(jnp.float32)
        b = b_ref[...].astype(jnp.float32)

        sends = []
        out_copies = [None, None]

        def finish(blk):
            for k in range(1, N_DEV):
                src = (my_pos + k) % N_DEV
                recv = pltpu.make_async_remote_copy(
                    src_ref=comm_ref.at[src, blk],
                    dst_ref=comm_ref.at[src, blk],
                    send_sem=send_sems.at[blk, k - 1],
                    recv_sem=recv_sems.at[src, blk],
                    device_id=(src,),
                    device_id_type=pl.DeviceIdType.MESH,
                )
                recv.wait_recv()
            tot = jnp.sum(comm_ref[:, blk], axis=0)
            mean = unpack(tot[0]) / n_global
            ex2 = unpack(tot[1]) / n_global
            inv = lax.rsqrt(ex2 - mean * mean + EPS)
            slot = blk % 2
            if out_copies[slot] is not None:
                out_copies[slot].wait()
            xb = xv[blk * mb:(blk + 1) * mb, :]
            ov[slot] = ((xb - mean) * inv * g + b).astype(ov.dtype)
            cp = pltpu.make_async_copy(
                ov.at[slot],
                out_hbm.at[pl.ds(blk * mb, mb), :],
                out_sems.at[slot],
            )
            cp.start()
            out_copies[slot] = cp

        for blk in range(B):
            in_copies[blk].wait()
            xb = xv[blk * mb:(blk + 1) * mb, :]
            s = jnp.sum(xb, axis=1)
            ss = jnp.sum(xb * xb, axis=1)
            comm_ref[my_pos, blk, 0] = s.reshape(rb, 128)
            comm_ref[my_pos, blk, 1] = ss.reshape(rb, 128)
            for k in range(1, N_DEV):
                peer = (my_pos + k) % N_DEV
                rdma = pltpu.make_async_remote_copy(
                    src_ref=comm_ref.at[my_pos, blk],
                    dst_ref=comm_ref.at[my_pos, blk],
                    send_sem=send_sems.at[blk, k - 1],
                    recv_sem=recv_sems.at[my_pos, blk],
                    device_id=(peer,),
                    device_id_type=pl.DeviceIdType.MESH,
                )
                rdma.start()
                sends.append(rdma)
        for blk in range(B):
            finish(blk)

        for cp in out_copies:
            if cp is not None:
                cp.wait()
        for r in sends:
            r.wait_send()

    return pl.pallas_call(
        body,
        out_shape=jax.ShapeDtypeStruct((m, n_local), jnp.bfloat16),
        in_specs=[
            pl.BlockSpec(memory_space=pltpu.MemorySpace.HBM),
            pl.BlockSpec(memory_space=pltpu.VMEM),
            pl.BlockSpec(memory_space=pltpu.VMEM),
        ],
        out_specs=pl.BlockSpec(memory_space=pltpu.MemorySpace.HBM),
        scratch_shapes=[
            pltpu.VMEM((m, n_local), jnp.float32),
            pltpu.VMEM((2, mb, n_local), jnp.bfloat16),
            pltpu.VMEM((N_DEV, B, 2, mb // 128, 128), jnp.float32),
            pltpu.SemaphoreType.DMA((B,)),
            pltpu.SemaphoreType.DMA((2,)),
            pltpu.SemaphoreType.DMA((B, N_DEV - 1)),
            pltpu.SemaphoreType.DMA((N_DEV, B)),
        ],
        compiler_params=pltpu.CompilerParams(
            collective_id=0, vmem_limit_bytes=100 * 1024 * 1024
        ),
    )(x, gamma.reshape(1, n_local), beta.reshape(1, n_local))
